# baseline (speedup 1.0000x reference)
import jax
import jax.numpy as jnp
from jax import lax
from jax.experimental import pallas as pl
from jax.experimental.pallas import tpu as pltpu
from jax.experimental.pallas import tpu_sc as plsc

_ENT_NUM = 1000000
_REL_NUM = 100
_DIM = 64
_BATCH = 16384

_NC = 2
_NS = 16
_NW = _NC * _NS
_BPW = _BATCH // _NW
_L = 16
_CHUNK = 128
_NCHUNK = _BPW // _CHUNK


def _score_body(head_hbm, rel_hbm, tail_hbm, ent_hbm, relemb_hbm, out_hbm,
                hidx_v, ridx_v, tidx_v, gh_v, gt_v, rel_v, hbuf, tbuf,
                out_v, sem):
    wid = lax.axis_index("s") * _NC + lax.axis_index("c")
    base = wid * _BPW

    pltpu.sync_copy(head_hbm.at[pl.ds(base, _BPW)], hidx_v)
    pltpu.sync_copy(rel_hbm.at[pl.ds(base, _BPW)], ridx_v)
    pltpu.sync_copy(tail_hbm.at[pl.ds(base, _BPW)], tidx_v)
    cp_rel = pltpu.async_copy(relemb_hbm, rel_v, sem)

    for k in range(_BPW // _L):
        c, off = k // (_CHUNK // _L), (k % (_CHUNK // _L)) * _L
        gh_v[c, pl.ds(off, _L)] = lax.shift_right_logical(
            hidx_v[pl.ds(k * _L, _L)], 1)
        gt_v[c, pl.ds(off, _L)] = lax.shift_right_logical(
            tidx_v[pl.ds(k * _L, _L)], 1)
    cp_rel.wait()

    lanes = lax.iota(jnp.int32, _L)
    one = jnp.int32(1)

    for c in range(_NCHUNK):
        cp_h = pltpu.async_copy(ent_hbm.at[gh_v.at[c]], hbuf, sem)
        cp_t = pltpu.async_copy(ent_hbm.at[gt_v.at[c]], tbuf, sem)
        cp_h.wait()
        cp_t.wait()

        def block(blk, carry, c=c):
            li0 = blk * _L
            i0 = c * _CHUNK + li0
            hv = hidx_v[pl.ds(i0, _L)]
            tv = tidx_v[pl.ds(i0, _L)]
            rv = ridx_v[pl.ds(i0, _L)]
            row = lanes + li0
            hcol = (hv & one) * _DIM
            tcol = (tv & one) * _DIM
            rrow = lax.shift_right_logical(rv, 1)
            rcol = (rv & one) * _DIM
            acc = jnp.zeros((_L,), jnp.float32)
            for d in range(_DIM):
                h = plsc.load_gather(hbuf, [row, hcol + d])
                r = plsc.load_gather(rel_v, [rrow, rcol + d])
                t = plsc.load_gather(tbuf, [row, tcol + d])
                acc = acc + jnp.abs(h + r - t)
            out_v[pl.ds(i0, _L)] = acc
            return carry

        lax.fori_loop(0, _CHUNK // _L, block, 0)

    pltpu.sync_copy(out_v, out_hbm.at[pl.ds(base, _BPW)])


@jax.jit
def _transe(head, rel, tail, ent2, relemb2):
    mesh = plsc.VectorSubcoreMesh(core_axis_name="c", subcore_axis_name="s")
    kern = pl.kernel(
        _score_body,
        mesh=mesh,
        out_type=jax.ShapeDtypeStruct((_BATCH,), jnp.float32),
        scratch_types=[
            pltpu.VMEM((_BPW,), jnp.int32),
            pltpu.VMEM((_BPW,), jnp.int32),
            pltpu.VMEM((_BPW,), jnp.int32),
            pltpu.VMEM((_NCHUNK, _CHUNK), jnp.int32),
            pltpu.VMEM((_NCHUNK, _CHUNK), jnp.int32),
            pltpu.VMEM((_REL_NUM // 2, 2 * _DIM), jnp.float32),
            pltpu.VMEM((_CHUNK, 2 * _DIM), jnp.float32),
            pltpu.VMEM((_CHUNK, 2 * _DIM), jnp.float32),
            pltpu.VMEM((_BPW,), jnp.float32),
            pltpu.SemaphoreType.DMA,
        ],
        compiler_params=pltpu.CompilerParams(needs_layout_passes=False),
    )
    return kern(head, rel, tail, ent2, relemb2)


def kernel(head, rel, tail, ent_embedding, rel_embedding):
    ent2 = ent_embedding.reshape(_ENT_NUM // 2, 2 * _DIM)
    relemb2 = rel_embedding.reshape(_REL_NUM // 2, 2 * _DIM)
    return _transe(head, rel, tail, ent2, relemb2)
